# baseline (speedup 1.0000x reference)
import functools

import jax
import jax.numpy as jnp
from jax import lax
from jax.experimental import pallas as pl
from jax.experimental.pallas import tpu as pltpu
from jax.experimental.pallas import tpu_sc as plsc

BATCH = 16384
HIST = 200
D = 64
B = BATCH * HIST
NC, NS = 2, 16
NW = NC * NS
BPW = B // NW
C = 1024
GPC = C // 128
NCH = BPW // C

_mesh = plsc.VectorSubcoreMesh(core_axis_name="c", subcore_axis_name="s")


@functools.partial(
    pl.kernel,
    mesh=_mesh,
    out_type=jax.ShapeDtypeStruct((B, D), jnp.float32),
    scratch_types=[
        pltpu.VMEM((GPC, 128), jnp.int32),
        pltpu.VMEM((C, D), jnp.float32),
        pltpu.SemaphoreType.DMA,
    ],
    compiler_params=pltpu.CompilerParams(use_tc_tiling_on_sc=False),
)
def _emb_gather(idx_hbm, table_hbm, out_hbm, idx_v, rows_v, sem):
    wid = lax.axis_index("s") * NC + lax.axis_index("c")
    base = wid * BPW

    def chunk(g, carry):
        row0 = base + g * C
        irow0 = pl.multiple_of(row0 // 128, 8)
        pltpu.sync_copy(idx_hbm.at[pl.ds(irow0, GPC)], idx_v)
        copies = []
        for j in range(GPC):
            copies.append(
                pltpu.async_copy(
                    table_hbm.at[idx_v.at[j]],
                    rows_v.at[pl.ds(j * 128, 128)],
                    sem,
                )
            )
        for cp in copies:
            cp.wait()
        pltpu.sync_copy(rows_v, out_hbm.at[pl.ds(row0, C)])
        return carry

    lax.fori_loop(0, NCH, chunk, 0)


def kernel(indices, table):
    idx2d = indices.reshape(B // 128, 128)
    out = _emb_gather(idx2d, table)
    return out.reshape(BATCH, HIST, D)


# speedup vs baseline: 1.0294x; 1.0294x over previous
import functools

import jax
import jax.numpy as jnp
from jax import lax
from jax.experimental import pallas as pl
from jax.experimental.pallas import tpu as pltpu
from jax.experimental.pallas import tpu_sc as plsc

BATCH = 16384
HIST = 200
D = 64
B = BATCH * HIST
NC, NS = 2, 16
NW = NC * NS
BPW = B // NW
C = 512
HCH = BPW // C
NSUP = BPW // (2 * C)

_mesh = plsc.VectorSubcoreMesh(core_axis_name="c", subcore_axis_name="s")


@functools.partial(
    pl.kernel,
    mesh=_mesh,
    out_type=jax.ShapeDtypeStruct((B, D), jnp.float32),
    scratch_types=[
        pltpu.VMEM((2, 8, 128), jnp.int32),
        pltpu.VMEM((2, C, D), jnp.float32),
        pltpu.SemaphoreType.DMA,
        pltpu.SemaphoreType.DMA,
        pltpu.SemaphoreType.DMA,
        pltpu.SemaphoreType.DMA,
        pltpu.SemaphoreType.DMA,
        pltpu.SemaphoreType.DMA,
    ],
    compiler_params=pltpu.CompilerParams(use_tc_tiling_on_sc=False),
)
def _emb_gather(idx_hbm, table_hbm, out_hbm, idx_v, rows_v,
                sin0, sin1, sout0, sout1, sidx0, sidx1):
    wid = lax.axis_index("s") * NC + lax.axis_index("c")
    base = wid * BPW
    ibase = base // 128
    sin = (sin0, sin1)
    sout = (sout0, sout1)
    sidx = (sidx0, sidx1)

    def fire_gathers(h, b, q, odd):
        for j in range(4):
            pltpu.async_copy(
                table_hbm.at[idx_v.at[q, 4 * odd + j]],
                rows_v.at[b, pl.ds(j * 128, 128)],
                sin[b],
            )

    def wait_gathers(b):
        pltpu.make_async_copy(out_hbm.at[pl.ds(0, C)], rows_v.at[b], sin[b]).wait()

    def fire_write(h, b):
        pltpu.async_copy(rows_v.at[b], out_hbm.at[pl.ds(base + h * C, C)], sout[b])

    def wait_write(b):
        pltpu.make_async_copy(rows_v.at[b], out_hbm.at[pl.ds(0, C)], sout[b]).wait()

    def fire_idx(s, q):
        off = pl.multiple_of(ibase + s * 8, 8)
        pltpu.async_copy(idx_hbm.at[pl.ds(off, 8)], idx_v.at[q], sidx[q])

    def wait_idx(q):
        pltpu.make_async_copy(idx_hbm.at[pl.ds(0, 8)], idx_v.at[q], sidx[q]).wait()

    fire_idx(0, 0)
    wait_idx(0)
    fire_gathers(0, 0, 0, 0)
    fire_idx(1, 1)
    fire_gathers(1, 1, 0, 1)
    wait_gathers(0)
    fire_write(0, 0)

    def body(u, carry):
        h = 2 + 4 * u
        wait_idx(1)
        wait_write(0)
        fire_gathers(h, 0, 1, 0)
        wait_gathers(1)
        fire_write(h - 1, 1)
        fire_idx(h // 2 + 1, 0)
        wait_write(1)
        fire_gathers(h + 1, 1, 1, 1)
        wait_gathers(0)
        fire_write(h, 0)
        wait_idx(0)
        wait_write(0)
        fire_gathers(h + 2, 0, 0, 0)
        wait_gathers(1)
        fire_write(h + 1, 1)
        fire_idx(h // 2 + 2, 1)
        wait_write(1)
        fire_gathers(h + 3, 1, 0, 1)
        wait_gathers(0)
        fire_write(h + 2, 0)
        return carry

    lax.fori_loop(0, (HCH - 4) // 4, body, 0)

    h = HCH - 2
    wait_idx(1)
    wait_write(0)
    fire_gathers(h, 0, 1, 0)
    wait_gathers(1)
    fire_write(h - 1, 1)
    wait_write(1)
    fire_gathers(h + 1, 1, 1, 1)
    wait_gathers(0)
    fire_write(h, 0)
    wait_gathers(1)
    fire_write(h + 1, 1)
    wait_write(0)
    wait_write(1)


def kernel(indices, table):
    idx2d = indices.reshape(B // 128, 128)
    out = _emb_gather(idx2d, table)
    return out.reshape(BATCH, HIST, D)
